# baseline (speedup 1.0000x reference)
import functools

import jax
import jax.numpy as jnp
from jax import lax
from jax.experimental import pallas as pl
from jax.experimental.pallas import tpu as pltpu
from jax.experimental.pallas import tpu_sc as plsc

CAPACITY = 1_000_000
ROW = 67
BATCH = 16384
PAD = 128

_NW = 32
_BPW = BATCH // _NW
_NCHUNK = 4
_CS = _BPW // _NCHUNK

_mesh = plsc.VectorSubcoreMesh(core_axis_name="c", subcore_axis_name="s")


@functools.partial(
    pl.kernel,
    mesh=_mesh,
    out_type=jax.ShapeDtypeStruct((BATCH, ROW), jnp.float32),
    scratch_types=[
        pltpu.VMEM((_BPW,), jnp.int32),
        [pltpu.VMEM((_CS, PAD), jnp.float32) for _ in range(2)],
        pltpu.VMEM((_BPW, ROW), jnp.float32),
        [pltpu.SemaphoreType.DMA for _ in range(2)],
    ],
    compiler_params=pltpu.CompilerParams(
        needs_layout_passes=False, skip_device_barrier=True),
)
def _sample(mem128, idx_hbm, out_hbm, idx_v, raw, out_v, sem):
    wid = lax.axis_index("s") * 2 + lax.axis_index("c")
    base = wid * _BPW
    pltpu.sync_copy(idx_hbm.at[pl.ds(base, _BPW)], idx_v)

    def fire(ch):
        return pltpu.async_copy(
            mem128.at[idx_v.at[pl.ds(ch * _CS, _CS)]], raw[ch % 2], sem[ch % 2])

    handles = {0: fire(0)}

    def compact_chunk(ch):
        buf = raw[ch % 2]

        def body(m, _):
            kk = ch * _CS + m
            for c in (0, 16, 32, 48, 51):
                out_v[kk, pl.ds(c, 16)] = buf[m, pl.ds(c, 16)]
            return _

        lax.fori_loop(0, _CS, body, 0, unroll=4)

    for ch in range(_NCHUNK):
        if ch + 1 < _NCHUNK:
            handles[ch + 1] = fire(ch + 1)
        handles[ch].wait()
        compact_chunk(ch)

    pltpu.sync_copy(out_v, out_hbm.at[pl.ds(base, _BPW)])


def kernel(memory, indices):
    wide = jnp.pad(memory, ((0, 0), (0, PAD - ROW)))
    return _sample(wide, indices)


# speedup vs baseline: 3.6915x; 3.6915x over previous
import functools

import jax
import jax.numpy as jnp
from jax import lax
from jax.experimental import pallas as pl
from jax.experimental.pallas import tpu as pltpu
from jax.experimental.pallas import tpu_sc as plsc

CAPACITY = 1_000_000
ROW = 67
BATCH = 16384

_NW = 32
_BPW = BATCH // _NW

_mesh = plsc.VectorSubcoreMesh(core_axis_name="c", subcore_axis_name="s")


@functools.partial(
    pl.kernel,
    mesh=_mesh,
    out_type=jax.ShapeDtypeStruct((BATCH, ROW), jnp.float32),
    scratch_types=[
        pltpu.VMEM((_BPW,), jnp.int32),
        pltpu.VMEM((_BPW, ROW), jnp.float32),
        [pltpu.SemaphoreType.DMA for _ in range(2)],
    ],
    compiler_params=pltpu.CompilerParams(
        needs_layout_passes=False, skip_device_barrier=True),
)
def _sample(mem_hbm, idx_hbm, out_hbm, idx_v, rows_v, sem):
    wid = lax.axis_index("s") * 2 + lax.axis_index("c")
    base = wid * _BPW
    pltpu.sync_copy(idx_hbm.at[pl.ds(base, _BPW)], idx_v)

    def fire(m, _):
        i16 = idx_v[pl.ds(m * 16, 16)]
        for l in range(16):
            k = m * 16 + l
            pltpu.async_copy(mem_hbm.at[i16[l]], rows_v.at[k], sem[l % 2])
        return _

    lax.fori_loop(0, _BPW // 16, fire, 0)

    def drain(m, _):
        pltpu.make_async_copy(mem_hbm.at[0], rows_v.at[0], sem[0]).wait()
        pltpu.make_async_copy(mem_hbm.at[0], rows_v.at[0], sem[1]).wait()
        return _

    lax.fori_loop(0, _BPW // 2, drain, 0)

    pltpu.sync_copy(rows_v, out_hbm.at[pl.ds(base, _BPW)])


def kernel(memory, indices):
    return _sample(memory, indices)
